# baseline (speedup 1.0000x reference)
import functools

import jax
import jax.numpy as jnp
from jax import lax
from jax.experimental import pallas as pl
from jax.experimental.pallas import tpu as pltpu
from jax.experimental.pallas import tpu_sc as plsc

D = 64
B = 4096
C = 100

NC = 2
NS = 16
NW = NC * NS
BPW = B // NW
L = 16
CG = (C + L - 1) // L
_STARTS = tuple(min(g * L, C - L) for g in range(CG))

_mesh = plsc.VectorSubcoreMesh(core_axis_name="c", subcore_axis_name="s")
_sc_params = pltpu.CompilerParams(use_tc_tiling_on_sc=False,
                                  needs_layout_passes=False)

CH = 16


@functools.partial(
    pl.kernel,
    mesh=_mesh,
    out_type=(
        jax.ShapeDtypeStruct((B, D), jnp.float32),
        jax.ShapeDtypeStruct((B, D), jnp.float32),
        jax.ShapeDtypeStruct((B, D), jnp.float32),
    ),
    scratch_types=[
        pltpu.VMEM((CH, C, 4), jnp.int32),
        pltpu.VMEM((BPW,), jnp.int32),
        pltpu.VMEM((BPW,), jnp.int32),
        pltpu.VMEM((BPW,), jnp.int32),
        pltpu.VMEM((BPW, D), jnp.float32),
        pltpu.VMEM((BPW, D), jnp.float32),
        pltpu.VMEM((BPW, D), jnp.float32),
        pltpu.SemaphoreType.DMA,
        pltpu.SemaphoreType.DMA,
        pltpu.SemaphoreType.DMA,
    ],
    compiler_params=_sc_params,
)
def _small_gathers(e_tab, r_tab, u_tab, data3,
                   e_out, r_out, u_out,
                   data_v, ie_v, ir_v, iu_v, e_v, r_v, u_v, se, sr, su):
    wid = lax.axis_index("s") * NC + lax.axis_index("c")
    base = wid * BPW
    lanes = lax.iota(jnp.int32, L)
    zz = jnp.zeros((L,), jnp.int32)
    for ch in range(BPW // CH):
        pltpu.sync_copy(data3.at[pl.ds(base + ch * CH, CH)], data_v)
        for t in range(CH // L):
            rv = lanes + L * t
            o = ch * CH + L * t
            ie_v[pl.ds(o, L)] = plsc.load_gather(data_v, [rv, zz, zz])
            ir_v[pl.ds(o, L)] = plsc.load_gather(data_v, [rv, zz, zz + 1])
            iu_v[pl.ds(o, L)] = plsc.load_gather(data_v, [rv, zz, zz + 2])
    ce = pltpu.async_copy(e_tab.at[ie_v], e_v, se)
    cr = pltpu.async_copy(r_tab.at[ir_v], r_v, sr)
    cu = pltpu.async_copy(u_tab.at[iu_v], u_v, su)
    ce.wait()
    cr.wait()
    cu.wait()
    pltpu.sync_copy(e_v, e_out.at[pl.ds(base, BPW)])
    pltpu.sync_copy(r_v, r_out.at[pl.ds(base, BPW)])
    pltpu.sync_copy(u_v, u_out.at[pl.ds(base, BPW)])


def _mlp_body(e_ref, r_ref, u_ref, w1_ref, b1_ref, w2_ref, b2_ref, s_ref):
    w1 = w1_ref[...]
    dn = (((1,), (1,)), ((), ()))
    h = lax.dot_general(e_ref[...], w1[:, :D], dn,
                        preferred_element_type=jnp.float32,
                        precision=lax.Precision.HIGHEST)
    h = h + lax.dot_general(r_ref[...], w1[:, D:], dn,
                            preferred_element_type=jnp.float32,
                            precision=lax.Precision.HIGHEST)
    h = jnp.maximum(h + b1_ref[...], 0.0)
    q = lax.dot_general(h, w2_ref[...], dn,
                        preferred_element_type=jnp.float32,
                        precision=lax.Precision.HIGHEST)
    s_ref[...] = q + b2_ref[...] + u_ref[...]


_MLP_BLK = B // 4

_mlp = pl.pallas_call(
    _mlp_body,
    grid=(4,),
    in_specs=[
        pl.BlockSpec((_MLP_BLK, D), lambda i: (i, 0)),
        pl.BlockSpec((_MLP_BLK, D), lambda i: (i, 0)),
        pl.BlockSpec((_MLP_BLK, D), lambda i: (i, 0)),
        pl.BlockSpec((D, 2 * D), lambda i: (0, 0)),
        pl.BlockSpec((1, D), lambda i: (0, 0)),
        pl.BlockSpec((D, D), lambda i: (0, 0)),
        pl.BlockSpec((1, D), lambda i: (0, 0)),
    ],
    out_specs=pl.BlockSpec((_MLP_BLK, D), lambda i: (i, 0)),
    out_shape=jax.ShapeDtypeStruct((B, D), jnp.float32),
)


NBUF = 8
CHF = 8
NCH = BPW // CHF


@functools.partial(
    pl.kernel,
    mesh=_mesh,
    out_type=jax.ShapeDtypeStruct((B, C), jnp.float32),
    scratch_types=[
        pltpu.VMEM((2, CHF, C, 4), jnp.int32),
        pltpu.VMEM((BPW, C), jnp.int32),
        pltpu.VMEM((BPW, D), jnp.float32),
        pltpu.VMEM((NBUF, C, D), jnp.float32),
        pltpu.VMEM((BPW, C), jnp.float32),
    ] + [pltpu.SemaphoreType.DMA] * (NBUF + 2),
    compiler_params=_sc_params,
)
def _fused_gather_dot(tab, data3, s, out,
                      datch, aidx_v, s_v, rows_v, out_v, *sems):
    dsems = sems[NBUF:]
    wid = lax.axis_index("s") * NC + lax.axis_index("c")
    base = wid * BPW
    pltpu.sync_copy(s.at[pl.ds(base, BPW)], s_v)

    lanes = lax.iota(jnp.int32, L)
    zero16 = jnp.zeros((L,), jnp.int32)

    colv = [jnp.minimum(lanes + L * t, C - 1) for t in range(CG)]
    three = jnp.full((L,), 3, jnp.int32)

    def extract(ch, buf):
        def ebody(b_l, carry):
            gb = ch * CHF + b_l
            row = jnp.full((L,), b_l, jnp.int32)
            for t in range(CG - 1):
                aidx_v[gb, pl.ds(L * t, L)] = plsc.load_gather(
                    datch.at[buf], [row, colv[t], three])
            vals = plsc.load_gather(datch.at[buf], [row, colv[CG - 1], three])
            plsc.store_scatter(aidx_v,
                               [jnp.full((L,), gb, jnp.int32), lanes + 96],
                               vals, mask=lanes < C - 96)
            return carry
        lax.fori_loop(0, CHF, ebody, 0)

    pltpu.async_copy(data3.at[pl.ds(base, CHF)], datch.at[0], dsems[0])
    for ch in range(NCH):
        buf = ch % 2
        if ch + 1 < NCH:
            pltpu.async_copy(data3.at[pl.ds(base + (ch + 1) * CHF, CHF)],
                             datch.at[1 - buf], dsems[1 - buf])
        pltpu.make_async_copy(data3.at[pl.ds(base + ch * CHF, CHF)],
                              datch.at[buf], dsems[buf]).wait()
        extract(ch, buf)

    cand64 = [(lanes + st) * D for st in _STARTS]
    NK = D // L

    def compute(b, rows):
        schunks = [s_v[b, pl.ds(L * k, L)] for k in range(NK)]

        def dbody(dd, accs):
            ddvec = jnp.full((L,), dd, jnp.int32)
            new = list(accs)
            for k in range(NK):
                sd = schunks[k].at[ddvec].get(mode="promise_in_bounds")
                col = jnp.full((L,), dd + L * k, jnp.int32)
                for g in range(CG):
                    v = plsc.load_gather(rows, [zero16, cand64[g] + col])
                    new[g] = new[g] + v * sd
            return tuple(new)

        accs = lax.fori_loop(
            0, L, dbody, tuple(jnp.zeros((L,), jnp.float32) for _ in range(CG)))
        for g in range(CG):
            out_v[b, pl.ds(_STARTS[g], L)] = accs[g]

    for j in range(NBUF):
        pltpu.async_copy(tab.at[aidx_v.at[j]], rows_v.at[j], sems[j])

    def body(i, carry):
        for j in range(NBUF):
            b = i * NBUF + j
            pltpu.make_async_copy(
                tab.at[aidx_v.at[b]], rows_v.at[j], sems[j]).wait()
            compute(b, rows_v.at[j])
            nxt = b + NBUF

            @pl.when(nxt < BPW)
            def _():
                pltpu.async_copy(tab.at[aidx_v.at[nxt]], rows_v.at[j], sems[j])
        return carry

    lax.fori_loop(0, BPW // NBUF, body, 0)
    pltpu.sync_copy(out_v, out.at[pl.ds(base, BPW)])


def kernel(data, e_table, r_table, u_table, W1, b1, W2, b2):
    data3 = data.astype(jnp.int32)
    e_emb, r_emb, u_emb = _small_gathers(e_table, r_table, u_table, data3)
    s = _mlp(e_emb, r_emb, u_emb, W1, b1.reshape(1, D), W2, b2.reshape(1, D))
    return _fused_gather_dot(e_table, data3, s)


# speedup vs baseline: 1.4951x; 1.4951x over previous
import functools

import jax
import jax.numpy as jnp
from jax import lax
from jax.experimental import pallas as pl
from jax.experimental.pallas import tpu as pltpu
from jax.experimental.pallas import tpu_sc as plsc

D = 64
B = 4096
C = 100

NC = 2
NS = 16
NW = NC * NS
BPW = B // NW
L = 16
CG = (C + L - 1) // L
_STARTS = tuple(min(g * L, C - L) for g in range(CG))

FPW = BPW * C
SLAB = 256
NSLAB = FPW // SLAB
RING = 4 * SLAB
RMASK = RING - 1

_mesh = plsc.VectorSubcoreMesh(core_axis_name="c", subcore_axis_name="s")
_sc_params = pltpu.CompilerParams(use_tc_tiling_on_sc=False,
                                  needs_layout_passes=False)


@functools.partial(
    pl.kernel,
    mesh=_mesh,
    out_type=(
        jax.ShapeDtypeStruct((B, D), jnp.float32),
        jax.ShapeDtypeStruct((B, D), jnp.float32),
        jax.ShapeDtypeStruct((B, D), jnp.float32),
    ),
    scratch_types=[
        pltpu.VMEM((BPW,), jnp.int32),
        pltpu.VMEM((BPW,), jnp.int32),
        pltpu.VMEM((BPW,), jnp.int32),
        pltpu.VMEM((BPW, D), jnp.float32),
        pltpu.VMEM((BPW, D), jnp.float32),
        pltpu.VMEM((BPW, D), jnp.float32),
        pltpu.SemaphoreType.DMA,
        pltpu.SemaphoreType.DMA,
        pltpu.SemaphoreType.DMA,
    ],
    compiler_params=_sc_params,
)
def _small_gathers(e_tab, r_tab, u_tab, ie, ir, iu,
                   e_out, r_out, u_out,
                   ie_v, ir_v, iu_v, e_v, r_v, u_v, se, sr, su):
    wid = lax.axis_index("s") * NC + lax.axis_index("c")
    base = wid * BPW
    pltpu.sync_copy(ie.at[pl.ds(base, BPW)], ie_v)
    pltpu.sync_copy(ir.at[pl.ds(base, BPW)], ir_v)
    pltpu.sync_copy(iu.at[pl.ds(base, BPW)], iu_v)
    ce = pltpu.async_copy(e_tab.at[ie_v], e_v, se)
    cr = pltpu.async_copy(r_tab.at[ir_v], r_v, sr)
    cu = pltpu.async_copy(u_tab.at[iu_v], u_v, su)
    ce.wait()
    cr.wait()
    cu.wait()
    pltpu.sync_copy(e_v, e_out.at[pl.ds(base, BPW)])
    pltpu.sync_copy(r_v, r_out.at[pl.ds(base, BPW)])
    pltpu.sync_copy(u_v, u_out.at[pl.ds(base, BPW)])


def _mlp_body(e_ref, r_ref, u_ref, w1_ref, b1_ref, w2_ref, b2_ref, s_ref):
    w1 = w1_ref[...]
    dn = (((1,), (1,)), ((), ()))
    h = lax.dot_general(e_ref[...], w1[:, :D], dn,
                        preferred_element_type=jnp.float32,
                        precision=lax.Precision.HIGHEST)
    h = h + lax.dot_general(r_ref[...], w1[:, D:], dn,
                            preferred_element_type=jnp.float32,
                            precision=lax.Precision.HIGHEST)
    h = jnp.maximum(h + b1_ref[...], 0.0)
    q = lax.dot_general(h, w2_ref[...], dn,
                        preferred_element_type=jnp.float32,
                        precision=lax.Precision.HIGHEST)
    s_ref[...] = q + b2_ref[...] + u_ref[...]


_MLP_BLK = B // 4

_mlp = pl.pallas_call(
    _mlp_body,
    grid=(4,),
    in_specs=[
        pl.BlockSpec((_MLP_BLK, D), lambda i: (i, 0)),
        pl.BlockSpec((_MLP_BLK, D), lambda i: (i, 0)),
        pl.BlockSpec((_MLP_BLK, D), lambda i: (i, 0)),
        pl.BlockSpec((D, 2 * D), lambda i: (0, 0)),
        pl.BlockSpec((1, D), lambda i: (0, 0)),
        pl.BlockSpec((D, D), lambda i: (0, 0)),
        pl.BlockSpec((1, D), lambda i: (0, 0)),
    ],
    out_specs=pl.BlockSpec((_MLP_BLK, D), lambda i: (i, 0)),
    out_shape=jax.ShapeDtypeStruct((B, D), jnp.float32),
)


@functools.partial(
    pl.kernel,
    mesh=_mesh,
    out_type=jax.ShapeDtypeStruct((B, C), jnp.float32),
    scratch_types=[
        pltpu.VMEM((NSLAB, SLAB), jnp.int32),
        pltpu.VMEM((BPW, D), jnp.float32),
        pltpu.VMEM((RING, D), jnp.float32),
        pltpu.VMEM((BPW, C), jnp.float32),
        pltpu.SemaphoreType.DMA,
        pltpu.SemaphoreType.DMA,
        pltpu.SemaphoreType.DMA,
        pltpu.SemaphoreType.DMA,
    ],
    compiler_params=_sc_params,
)
def _fused_gather_dot(tab, ia2, s, out,
                      aidx_v, s_v, ring, out_v, *sems):
    wid = lax.axis_index("s") * NC + lax.axis_index("c")
    base = wid * BPW
    pltpu.sync_copy(ia2.at[pl.ds(wid * NSLAB, NSLAB)], aidx_v)
    pltpu.sync_copy(s.at[pl.ds(base, BPW)], s_v)

    lanes = lax.iota(jnp.int32, L)
    zero16 = jnp.zeros((L,), jnp.int32)
    NK = D // L

    def slab_copy(j, slot):
        return pltpu.make_async_copy(
            tab.at[aidx_v.at[j]],
            ring.at[pl.ds(slot * SLAB, SLAB)],
            sems[slot])

    def compute(b):
        schunks = [s_v[b, pl.ds(L * k, L)] for k in range(NK)]
        fb = b * C
        cand64 = [((lanes + (st + fb)) & RMASK) * D for st in _STARTS]

        def dbody(dd, accs):
            ddvec = jnp.full((L,), dd, jnp.int32)
            new = list(accs)
            for k in range(NK):
                sd = schunks[k].at[ddvec].get(mode="promise_in_bounds")
                col = jnp.full((L,), dd + L * k, jnp.int32)
                for g in range(CG):
                    v = plsc.load_gather(ring, [zero16, cand64[g] + col])
                    new[g] = new[g] + v * sd
            return tuple(new)

        accs = lax.fori_loop(
            0, L, dbody, tuple(jnp.zeros((L,), jnp.float32) for _ in range(CG)))
        for g in range(CG):
            out_v[b, pl.ds(_STARTS[g], L)] = accs[g]

    slab_copy(0, 0).start()
    slab_copy(1, 1).start()

    def body(b, jprev):
        jneed = (b * C + C - 1) >> 8

        @pl.when(jneed != jprev)
        def _():
            for slot in range(4):
                @pl.when((jneed & 3) == slot)
                def _():
                    slab_copy(jneed, slot).wait()
                    nxt = jneed + 2

                    @pl.when(nxt < NSLAB)
                    def _():
                        slab_copy(nxt, (slot + 2) & 3).start()

        compute(b)
        return jneed

    lax.fori_loop(0, BPW, body, jnp.int32(-1))
    pltpu.sync_copy(out_v, out.at[pl.ds(base, BPW)])


def kernel(data, e_table, r_table, u_table, W1, b1, W2, b2):
    data = data.astype(jnp.int32)
    ie = data[:, 0, 0]
    ir = data[:, 0, 1]
    iu = data[:, 0, 2]
    ia2 = data[:, :, 3].reshape(B * C // SLAB, SLAB)
    e_emb, r_emb, u_emb = _small_gathers(e_table, r_table, u_table, ie, ir, iu)
    s = _mlp(e_emb, r_emb, u_emb, W1, b1.reshape(1, D), W2, b2.reshape(1, D))
    return _fused_gather_dot(e_table, ia2, s)
